# baseline (speedup 1.0000x reference)
import jax
import jax.numpy as jnp
from jax import lax
from jax.experimental import pallas as pl
from jax.experimental.pallas import tpu as pltpu
from jax.experimental.pallas import tpu_sc as plsc

N = 10000
D = 128
R = 3
E = 100000

NC = 2
NS = 16
NW = NC * NS
B = 128
EDGES = R * E
K = -(-EDGES // (NW * B))
KL = (K + 7) // 8 * 8
PER_W = K * B
E_PAD = PER_W * NW
E_LAYOUT = NW * KL * B
ACC_ROWS = 10240
ZR = ACC_ROWS // NS


def _relation_mm_body(x_ref, w_ref, y_ref):
    y_ref[...] = jax.lax.dot(
        x_ref[...], w_ref[0],
        precision=jax.lax.Precision.HIGHEST,
        preferred_element_type=jnp.float32,
    )


def _relation_table(x, weight):
    return pl.pallas_call(
        _relation_mm_body,
        grid=(R,),
        in_specs=[
            pl.BlockSpec((N, D), lambda r: (0, 0)),
            pl.BlockSpec((1, D, D), lambda r: (r, 0, 0)),
        ],
        out_specs=pl.BlockSpec((N, D), lambda r: (r, 0)),
        out_shape=jax.ShapeDtypeStruct((R * N, D), jnp.float32),
    )(x, weight)


def _unpack_issue(pk_v, j, src_sm, dst_sm, y_hbm, rows, sem):
    for t in range(B // 16):
        v = pk_v[j, pl.ds(t * 16, 16)]
        src_sm[pl.ds(t * 16, 16)] = v & 0x7FFF
        dst_sm[pl.ds(t * 16, 16)] = v >> 15
    pltpu.async_copy(y_hbm.at[src_sm], rows, sem)


def _sc_scatter_body(y_hbm, pk_hbm, zeros_hbm, out_hbm,
                     pk_v, src0, dst0, src1, dst1, rows0, rows1, acc,
                     sem0, sem1, sems0, sems1):
    cid = lax.axis_index("c")
    sid = lax.axis_index("s")
    wid = sid * NC + cid
    pltpu.sync_copy(zeros_hbm.at[pl.ds(sid * ZR, ZR)],
                    acc.at[pl.ds(sid * ZR, ZR)])
    pltpu.sync_copy(pk_hbm.at[wid], pk_v)
    plsc.subcore_barrier()

    _unpack_issue(pk_v, 0, src0, dst0, y_hbm, rows0, sem0)
    _unpack_issue(pk_v, 1, src1, dst1, y_hbm, rows1, sem1)

    @pl.loop(0, K, step=2)
    def _(j):
        pltpu.make_async_copy(y_hbm.at[src0], rows0, sem0).wait()
        s0 = pltpu.async_copy(rows0, acc.at[dst0], sems0, add=True)
        pltpu.make_async_copy(y_hbm.at[src1], rows1, sem1).wait()
        s1 = pltpu.async_copy(rows1, acc.at[dst1], sems1, add=True)
        s0.wait()

        @pl.when(j + 2 < K)
        def _():
            _unpack_issue(pk_v, j + 2, src0, dst0, y_hbm, rows0, sem0)

        s1.wait()

        @pl.when(j + 3 < K)
        def _():
            _unpack_issue(pk_v, j + 3, src1, dst1, y_hbm, rows1, sem1)

    plsc.subcore_barrier()
    pltpu.sync_copy(acc.at[pl.ds(sid * ZR, ZR)],
                    out_hbm.at[cid, pl.ds(sid * ZR, ZR)])


def _sc_scatter(y, pk_arr, zeros):
    mesh = plsc.VectorSubcoreMesh(core_axis_name="c", subcore_axis_name="s")
    kern = pl.kernel(
        _sc_scatter_body,
        mesh=mesh,
        out_type=jax.ShapeDtypeStruct((NC, ACC_ROWS, D), jnp.float32),
        scratch_types=[
            pltpu.VMEM((KL, B), jnp.int32),
            pltpu.VMEM((B,), jnp.int32),
            pltpu.VMEM((B,), jnp.int32),
            pltpu.VMEM((B,), jnp.int32),
            pltpu.VMEM((B,), jnp.int32),
            pltpu.VMEM((B, D), jnp.float32),
            pltpu.VMEM((B, D), jnp.float32),
            pltpu.VMEM_SHARED((ACC_ROWS, D), jnp.float32),
            pltpu.SemaphoreType.DMA,
            pltpu.SemaphoreType.DMA,
            pltpu.SemaphoreType.DMA,
            pltpu.SemaphoreType.DMA,
        ],
    )
    return kern(y, pk_arr, zeros)


def _final_body(x_ref, lw_ref, p0_ref, p1_ref, out_ref):
    out_ref[...] = (
        jax.lax.dot(x_ref[...], lw_ref[...],
                    precision=jax.lax.Precision.HIGHEST,
                    preferred_element_type=jnp.float32)
        + p0_ref[0] + p1_ref[0]
    )


def _final(x, loop_weight, parts):
    bm = 2000
    nb = N // bm
    return pl.pallas_call(
        _final_body,
        grid=(nb,),
        in_specs=[
            pl.BlockSpec((bm, D), lambda i: (i, 0)),
            pl.BlockSpec((D, D), lambda i: (0, 0)),
            pl.BlockSpec((1, bm, D), lambda i: (0, i, 0)),
            pl.BlockSpec((1, bm, D), lambda i: (1, i, 0)),
        ],
        out_specs=pl.BlockSpec((bm, D), lambda i: (i, 0)),
        out_shape=jax.ShapeDtypeStruct((N, D), jnp.float32),
    )(x, loop_weight, parts, parts)


def kernel(x, edge_index, weight, loop_weight):
    src = edge_index[:, 1, :]
    dst = edge_index[:, 0, :]
    src_flat = (src + jnp.arange(R, dtype=jnp.int32)[:, None] * N).reshape(-1)
    dst_flat = dst.reshape(-1)
    pad = E_PAD - EDGES
    src_flat = jnp.concatenate([src_flat, jnp.zeros((pad,), jnp.int32)])
    dst_flat = jnp.concatenate([dst_flat, jnp.full((pad,), N, jnp.int32)])
    packed = src_flat | (dst_flat << 15)
    pk_arr = jnp.pad(packed.reshape(NW, K, B),
                     ((0, 0), (0, KL - K), (0, 0)),
                     constant_values=N << 15)
    zeros = jnp.zeros((ACC_ROWS, D), jnp.float32)

    y = _relation_table(x, weight)
    parts = _sc_scatter(y, pk_arr, zeros)
    return _final(x, loop_weight, parts)


# speedup vs baseline: 1.1860x; 1.1860x over previous
import jax
import jax.numpy as jnp
from jax import lax
from jax.experimental import pallas as pl
from jax.experimental.pallas import tpu as pltpu
from jax.experimental.pallas import tpu_sc as plsc

N = 10000
D = 128
R = 3
E = 100000

NC = 2
NS = 16
NW = NC * NS
B = 128
EDGES = R * E
K0 = 104
K1 = 44
KL = (K0 + 7) // 8 * 8
E_PAD = NS * (K0 + K1) * B
ACC_ROWS = 10240
ZR = ACC_ROWS // NS


def _relation_mm_body(x_ref, w_ref, y_ref):
    y_ref[...] = jax.lax.dot(
        x_ref[...], w_ref[0],
        precision=jax.lax.Precision.HIGHEST,
        preferred_element_type=jnp.float32,
    )


def _relation_table(x, weight):
    return pl.pallas_call(
        _relation_mm_body,
        grid=(R,),
        in_specs=[
            pl.BlockSpec((N, D), lambda r: (0, 0)),
            pl.BlockSpec((1, D, D), lambda r: (r, 0, 0)),
        ],
        out_specs=pl.BlockSpec((N, D), lambda r: (r, 0)),
        out_shape=jax.ShapeDtypeStruct((R * N, D), jnp.float32),
    )(x, weight)


def _unpack_issue(pk_v, j, src_sm, dst_sm, y_hbm, rows, sem):
    for t in range(B // 16):
        v = pk_v[j, pl.ds(t * 16, 16)]
        src_sm[pl.ds(t * 16, 16)] = v & 0x7FFF
        dst_sm[pl.ds(t * 16, 16)] = v >> 15
    pltpu.async_copy(y_hbm.at[src_sm], rows, sem)


def _sc_scatter_body(y_hbm, pk_hbm, zeros_hbm, out_hbm,
                     pk_v, src0, dst0, src1, dst1, rows0, rows1, acc,
                     sem0, sem1):
    cid = lax.axis_index("c")
    sid = lax.axis_index("s")
    wid = sid * NC + cid
    pltpu.sync_copy(zeros_hbm.at[pl.ds(sid * ZR, ZR)],
                    acc.at[pl.ds(sid * ZR, ZR)])
    pltpu.sync_copy(pk_hbm.at[wid], pk_v)
    plsc.subcore_barrier()

    k_lim = jnp.where(cid == 0, K0, K1)

    _unpack_issue(pk_v, 0, src0, dst0, y_hbm, rows0, sem0)
    _unpack_issue(pk_v, 1, src1, dst1, y_hbm, rows1, sem1)

    @pl.loop(0, k_lim, step=2)
    def _(j):
        pltpu.make_async_copy(y_hbm.at[src0], rows0, sem0).wait()
        pltpu.sync_copy(rows0, acc.at[dst0], add=True)

        @pl.when(j + 2 < k_lim)
        def _():
            _unpack_issue(pk_v, j + 2, src0, dst0, y_hbm, rows0, sem0)

        pltpu.make_async_copy(y_hbm.at[src1], rows1, sem1).wait()
        pltpu.sync_copy(rows1, acc.at[dst1], add=True)

        @pl.when(j + 3 < k_lim)
        def _():
            _unpack_issue(pk_v, j + 3, src1, dst1, y_hbm, rows1, sem1)

    plsc.subcore_barrier()
    pltpu.sync_copy(acc.at[pl.ds(sid * ZR, ZR)],
                    out_hbm.at[cid, pl.ds(sid * ZR, ZR)])


def _sc_scatter(y, pk_arr, zeros):
    mesh = plsc.VectorSubcoreMesh(core_axis_name="c", subcore_axis_name="s")
    kern = pl.kernel(
        _sc_scatter_body,
        mesh=mesh,
        out_type=jax.ShapeDtypeStruct((NC, ACC_ROWS, D), jnp.float32),
        scratch_types=[
            pltpu.VMEM((KL, B), jnp.int32),
            pltpu.VMEM((B,), jnp.int32),
            pltpu.VMEM((B,), jnp.int32),
            pltpu.VMEM((B,), jnp.int32),
            pltpu.VMEM((B,), jnp.int32),
            pltpu.VMEM((B, D), jnp.float32),
            pltpu.VMEM((B, D), jnp.float32),
            pltpu.VMEM_SHARED((ACC_ROWS, D), jnp.float32),
            pltpu.SemaphoreType.DMA,
            pltpu.SemaphoreType.DMA,
        ],
    )
    return kern(y, pk_arr, zeros)


def _final_body(x_ref, lw_ref, p0_ref, p1_ref, out_ref):
    out_ref[...] = (
        jax.lax.dot(x_ref[...], lw_ref[...],
                    precision=jax.lax.Precision.HIGHEST,
                    preferred_element_type=jnp.float32)
        + p0_ref[0] + p1_ref[0]
    )


def _final(x, loop_weight, parts):
    bm = 2000
    nb = N // bm
    return pl.pallas_call(
        _final_body,
        grid=(nb,),
        in_specs=[
            pl.BlockSpec((bm, D), lambda i: (i, 0)),
            pl.BlockSpec((D, D), lambda i: (0, 0)),
            pl.BlockSpec((1, bm, D), lambda i: (0, i, 0)),
            pl.BlockSpec((1, bm, D), lambda i: (1, i, 0)),
        ],
        out_specs=pl.BlockSpec((bm, D), lambda i: (i, 0)),
        out_shape=jax.ShapeDtypeStruct((N, D), jnp.float32),
    )(x, loop_weight, parts, parts)


def kernel(x, edge_index, weight, loop_weight):
    src = edge_index[:, 1, :]
    dst = edge_index[:, 0, :]
    src_flat = (src + jnp.arange(R, dtype=jnp.int32)[:, None] * N).reshape(-1)
    dst_flat = dst.reshape(-1)
    pad = E_PAD - EDGES
    src_flat = jnp.concatenate([src_flat, jnp.zeros((pad,), jnp.int32)])
    dst_flat = jnp.concatenate([dst_flat, jnp.full((pad,), N, jnp.int32)])
    packed = src_flat | (dst_flat << 15)
    pk0 = packed[: NS * K0 * B].reshape(NS, K0, B)
    pk1 = packed[NS * K0 * B:].reshape(NS, K1, B)
    pk0 = jnp.pad(pk0, ((0, 0), (0, KL - K0), (0, 0)),
                  constant_values=N << 15)
    pk1 = jnp.pad(pk1, ((0, 0), (0, KL - K1), (0, 0)),
                  constant_values=N << 15)
    pk_arr = jnp.stack([pk0, pk1], axis=1).reshape(NW, KL, B)
    zeros = jnp.zeros((ACC_ROWS, D), jnp.float32)

    y = _relation_table(x, weight)
    parts = _sc_scatter(y, pk_arr, zeros)
    return _final(x, loop_weight, parts)
